# Initial kernel scaffold; baseline (speedup 1.0000x reference)
import jax
import jax.numpy as jnp
from jax.experimental import pallas as pl


def kernel(hidden_states, router_w, correction_bias, w_gate, w_up, w_down):
    raise NotImplementedError("write your pallas kernel here")


# speedup vs baseline: 1.1594x; 1.1594x over previous
import jax
import jax.numpy as jnp
from jax.experimental import pallas as pl
from jax.experimental.pallas import tpu as pltpu

_TOP_K = 2
_ROUTED_SCALING = 1.0
_BF = 512


def _moe_tc_kernel(x_ref, rw_ref, bias_ref, wg_ref, wu_ref, wd_ref,
                   out_ref, combine_ref):
    e = pl.program_id(0)
    f = pl.program_id(1)
    T, E = combine_ref.shape

    @pl.when((e == 0) & (f == 0))
    def _router():
        x = x_ref[...]
        logits = jax.lax.dot_general(
            x, rw_ref[...], (((1,), (1,)), ((), ())),
            preferred_element_type=jnp.float32)
        m = jnp.max(logits, axis=1, keepdims=True)
        ex = jnp.exp(logits - m)
        scores = ex / jnp.sum(ex, axis=1, keepdims=True)
        biased = scores + bias_ref[...]
        eidx = jax.lax.broadcasted_iota(jnp.int32, (T, E), 1)
        m1 = jnp.max(biased, axis=1, keepdims=True)
        i1 = jnp.min(jnp.where(biased == m1, eidx, E), axis=1, keepdims=True)
        sel1 = eidx == i1
        masked = jnp.where(sel1, -jnp.inf, biased)
        m2 = jnp.max(masked, axis=1, keepdims=True)
        i2 = jnp.min(jnp.where(masked == m2, eidx, E), axis=1, keepdims=True)
        sel2 = eidx == i2
        combine_ref[...] = jnp.where(sel1 | sel2, scores, 0.0) * _ROUTED_SCALING
        out_ref[...] = jnp.zeros_like(out_ref)

    x = x_ref[...]
    xg = jnp.dot(x, wg_ref[0], preferred_element_type=jnp.float32)
    xu = jnp.dot(x, wu_ref[0], preferred_element_type=jnp.float32)
    h = (xg * jax.nn.sigmoid(xg)) * xu
    eidx = jax.lax.broadcasted_iota(jnp.int32, (T, E), 1)
    ccol = jnp.sum(jnp.where(eidx == e, combine_ref[...], 0.0),
                   axis=1, keepdims=True)
    out_ref[...] += jnp.dot(h * ccol, wd_ref[0],
                            preferred_element_type=jnp.float32)


def kernel(hidden_states, router_w, correction_bias, w_gate, w_up, w_down):
    T, H = hidden_states.shape
    E, _, I = w_gate.shape
    nf = I // _BF
    bias2d = correction_bias.reshape(1, E)
    return pl.pallas_call(
        _moe_tc_kernel,
        grid=(E, nf),
        in_specs=[
            pl.BlockSpec((T, H), lambda e, f: (0, 0)),
            pl.BlockSpec((E, H), lambda e, f: (0, 0)),
            pl.BlockSpec((1, E), lambda e, f: (0, 0)),
            pl.BlockSpec((1, H, _BF), lambda e, f: (e, 0, f)),
            pl.BlockSpec((1, H, _BF), lambda e, f: (e, 0, f)),
            pl.BlockSpec((1, _BF, H), lambda e, f: (e, f, 0)),
        ],
        out_specs=pl.BlockSpec((T, H), lambda e, f: (0, 0)),
        out_shape=jax.ShapeDtypeStruct((T, H), jnp.float32),
        scratch_shapes=[pltpu.VMEM((T, E), jnp.float32)],
        compiler_params=pltpu.CompilerParams(
            dimension_semantics=("arbitrary", "arbitrary"),
        ),
    )(hidden_states, router_w, bias2d, w_gate, w_up, w_down)


# speedup vs baseline: 1.2242x; 1.0559x over previous
import jax
import jax.numpy as jnp
from jax.experimental import pallas as pl
from jax.experimental.pallas import tpu as pltpu

_TOP_K = 2
_ROUTED_SCALING = 1.0
_BF = 1024


def _moe_tc_kernel(x_ref, rw_ref, bias_ref, wg_ref, wu_ref, wd_ref,
                   out_ref, combine_ref):
    e = pl.program_id(0)
    f = pl.program_id(1)
    T, E = combine_ref.shape

    @pl.when((e == 0) & (f == 0))
    def _router():
        x = x_ref[...]
        logits = jax.lax.dot_general(
            x, rw_ref[...], (((1,), (1,)), ((), ())),
            preferred_element_type=jnp.float32)
        m = jnp.max(logits, axis=1, keepdims=True)
        ex = jnp.exp(logits - m)
        scores = ex / jnp.sum(ex, axis=1, keepdims=True)
        biased = scores + bias_ref[...]
        eidx = jax.lax.broadcasted_iota(jnp.int32, (T, E), 1)
        m1 = jnp.max(biased, axis=1, keepdims=True)
        i1 = jnp.min(jnp.where(biased == m1, eidx, E), axis=1, keepdims=True)
        sel1 = eidx == i1
        masked = jnp.where(sel1, -jnp.inf, biased)
        m2 = jnp.max(masked, axis=1, keepdims=True)
        i2 = jnp.min(jnp.where(masked == m2, eidx, E), axis=1, keepdims=True)
        sel2 = eidx == i2
        combine_ref[...] = jnp.where(sel1 | sel2, scores, 0.0) * _ROUTED_SCALING
        out_ref[...] = jnp.zeros_like(out_ref)

    x = x_ref[...]
    xg = jnp.dot(x, wg_ref[0], preferred_element_type=jnp.float32)
    xu = jnp.dot(x, wu_ref[0], preferred_element_type=jnp.float32)
    h = (xg * jax.nn.sigmoid(xg)) * xu
    eidx = jax.lax.broadcasted_iota(jnp.int32, (T, E), 1)
    ccol = jnp.sum(jnp.where(eidx == e, combine_ref[...], 0.0),
                   axis=1, keepdims=True)
    out_ref[...] += jnp.dot(h * ccol, wd_ref[0],
                            preferred_element_type=jnp.float32)


def kernel(hidden_states, router_w, correction_bias, w_gate, w_up, w_down):
    T, H = hidden_states.shape
    E, _, I = w_gate.shape
    nf = I // _BF
    bias2d = correction_bias.reshape(1, E)
    return pl.pallas_call(
        _moe_tc_kernel,
        grid=(E, nf),
        in_specs=[
            pl.BlockSpec((T, H), lambda e, f: (0, 0)),
            pl.BlockSpec((E, H), lambda e, f: (0, 0)),
            pl.BlockSpec((1, E), lambda e, f: (0, 0)),
            pl.BlockSpec((1, H, _BF), lambda e, f: (e, 0, f)),
            pl.BlockSpec((1, H, _BF), lambda e, f: (e, 0, f)),
            pl.BlockSpec((1, _BF, H), lambda e, f: (e, f, 0)),
        ],
        out_specs=pl.BlockSpec((T, H), lambda e, f: (0, 0)),
        out_shape=jax.ShapeDtypeStruct((T, H), jnp.float32),
        scratch_shapes=[pltpu.VMEM((T, E), jnp.float32)],
        compiler_params=pltpu.CompilerParams(
            dimension_semantics=("arbitrary", "arbitrary"),
        ),
    )(hidden_states, router_w, bias2d, w_gate, w_up, w_down)
